# Initial kernel scaffold; baseline (speedup 1.0000x reference)
import jax
import jax.numpy as jnp
from jax.experimental import pallas as pl


def kernel(x, speaker_ids, codebooks):
    raise NotImplementedError("write your pallas kernel here")


# speedup vs baseline: 30.0732x; 30.0732x over previous
import jax
import jax.numpy as jnp
from jax.experimental import pallas as pl
from jax.experimental.pallas import tpu as pltpu

_B = 16
_C = 64
_L = 2048
_K = 8192
_TOPK = 4
_L_TILE = 256


def _vq_kernel(sid_ref, x_ref, cb_ref, out_ref):
    x = x_ref[0]
    norm = jnp.sqrt(jnp.sum(x * x, axis=0, keepdims=True))
    q = x / jnp.maximum(norm, 1e-6)
    codes = cb_ref[0]
    sim = jax.lax.dot_general(
        q, codes, (((0,), (1,)), ((), ())),
        preferred_element_type=jnp.float32)
    iota = jax.lax.broadcasted_iota(jnp.int32, (_L_TILE, _K), 1)
    work = sim
    mask_acc = jnp.zeros((_L_TILE, _K), jnp.float32)
    for _ in range(_TOPK):
        m = jnp.max(work, axis=1, keepdims=True)
        first = jnp.min(jnp.where(work == m, iota, _K), axis=1, keepdims=True)
        sel = iota == first
        mask_acc = mask_acc + sel.astype(jnp.float32)
        work = jnp.where(sel, -jnp.inf, work)
    out = jax.lax.dot_general(
        codes, mask_acc, (((0,), (1,)), ((), ())),
        preferred_element_type=jnp.float32)
    out_ref[0] = out * (1.0 / _TOPK)


def kernel(x, speaker_ids, codebooks):
    sids = speaker_ids.astype(jnp.int32)
    grid = (_B, _L // _L_TILE)
    grid_spec = pltpu.PrefetchScalarGridSpec(
        num_scalar_prefetch=1,
        grid=grid,
        in_specs=[
            pl.BlockSpec((1, _C, _L_TILE), lambda b, l, sid: (b, 0, l)),
            pl.BlockSpec((1, _K, _C), lambda b, l, sid: (sid[b], 0, 0)),
        ],
        out_specs=pl.BlockSpec((1, _C, _L_TILE), lambda b, l, sid: (b, 0, l)),
    )
    out = pl.pallas_call(
        _vq_kernel,
        grid_spec=grid_spec,
        out_shape=jax.ShapeDtypeStruct((_B, _C, _L), jnp.float32),
    )(sids, x, codebooks)
    return out.astype(x.dtype)
